# baseline (speedup 1.0000x reference)
import jax
import jax.numpy as jnp
from jax.experimental import pallas as pl
from jax.experimental.pallas import tpu as pltpu

_WIDTH = 2048
_KPAD = 2176
_NE = 64
_BM = 1024
_NROWS = 8192
_NSTEP = _NROWS // _BM
_NBUF = 4


def _router_body(x_hbm, w_ref, b_ref, off_ref, o_ref, buf, sems):
    wfull = jnp.concatenate(
        [w_ref[...], jnp.zeros((_NE, _KPAD - _WIDTH), jnp.float32)], axis=1
    )
    wp = pltpu.roll(wfull, off_ref[0], axis=1)
    rows = jax.lax.broadcasted_iota(jnp.int32, (_NE, _NE), 0)
    cols = jax.lax.broadcasted_iota(jnp.int32, (_NE, _NE), 1)
    eye = jnp.where(rows == cols, 1.0, 0.0).astype(jnp.float32)
    b_col = jax.lax.dot_general(
        eye, b_ref[...],
        dimension_numbers=(((1,), (1,)), ((), ())),
        preferred_element_type=jnp.float32,
    )

    def copy(i):
        return pltpu.make_async_copy(
            x_hbm.at[pl.ds(i * _BM, _BM), pl.ds(0, _KPAD)],
            buf.at[i % _NBUF],
            sems.at[i % _NBUF],
        )

    for i in range(_NBUF):
        copy(i).start()
    for i in range(_NSTEP):
        copy(i).wait()
        acc = jax.lax.dot_general(
            wp, buf[i % _NBUF],
            dimension_numbers=(((1,), (1,)), ((), ())),
            preferred_element_type=jnp.float32,
        )
        o_ref[:, i * _BM:(i + 1) * _BM] = acc + b_col
        if i + _NBUF < _NSTEP:
            copy(i + _NBUF).start()


def kernel(x, A, W, b):
    n = x.shape[0]
    a32 = A.astype(jnp.int32) if hasattr(A, "astype") else jnp.int32(A)
    off = jnp.reshape(a32 - _WIDTH, (1,))
    b2 = b.reshape(1, _NE)

    out_t = pl.pallas_call(
        _router_body,
        in_specs=[
            pl.BlockSpec(memory_space=pltpu.MemorySpace.HBM),
            pl.BlockSpec((_NE, _WIDTH), lambda: (0, 0)),
            pl.BlockSpec((1, _NE), lambda: (0, 0)),
            pl.BlockSpec(memory_space=pltpu.MemorySpace.SMEM),
        ],
        out_specs=pl.BlockSpec((_NE, n), lambda: (0, 0)),
        out_shape=jax.ShapeDtypeStruct((_NE, n), jnp.float32),
        scratch_shapes=[
            pltpu.VMEM((_NBUF, _BM, _KPAD), jnp.float32),
            pltpu.SemaphoreType.DMA((_NBUF,)),
        ],
    )(x, W, b2, off)
    return out_t.T
